# Initial kernel scaffold; baseline (speedup 1.0000x reference)
import jax
import jax.numpy as jnp
from jax.experimental import pallas as pl


def kernel(pred_logits):
    raise NotImplementedError("write your pallas kernel here")


# speedup vs baseline: 11.2744x; 11.2744x over previous
import jax
import jax.numpy as jnp
from jax.experimental import pallas as pl
from jax.experimental.pallas import tpu as pltpu

_BR = 2048


def _body(x_ref, o_ref):
    @pl.when(pl.program_id(0) == 0)
    def _init():
        o_ref[...] = jnp.zeros_like(o_ref)

    x = x_ref[...]
    sp = jnp.maximum(x, 0.0) + jnp.log1p(jnp.exp(-jnp.abs(x)))
    sp_sum = jnp.sum(sp)

    m1 = jnp.max(x, axis=1, keepdims=True)
    eq = x == m1
    cnt = jnp.sum(eq.astype(jnp.float32), axis=1, keepdims=True)
    neg_inf = jnp.float32(-jnp.inf)
    m2_strict = jnp.max(jnp.where(eq, neg_inf, x), axis=1, keepdims=True)
    m2 = jnp.where(cnt > 1.0, m1, m2_strict)
    gate = jax.nn.sigmoid(m1) > 0.5
    contrib = jnp.where(gate, m1 + m2, 0.0)
    o_ref[...] += sp_sum - jnp.sum(contrib)


def kernel(pred_logits):
    N, T = pred_logits.shape
    K = 64
    x2 = pred_logits.reshape(N * K, K)
    rows = N * K
    grid = rows // _BR
    out = pl.pallas_call(
        _body,
        grid=(grid,),
        in_specs=[pl.BlockSpec((_BR, K), lambda i: (i, 0))],
        out_specs=pl.BlockSpec((1, 1), lambda i: (0, 0)),
        out_shape=jax.ShapeDtypeStruct((1, 1), jnp.float32),
    )(x2)
    return out[0, 0]


# speedup vs baseline: 13.7331x; 1.2181x over previous
import jax
import jax.numpy as jnp
from jax.experimental import pallas as pl
from jax.experimental.pallas import tpu as pltpu

_BR = 256


def _body(x_ref, o_ref):
    @pl.when(pl.program_id(0) == 0)
    def _init():
        o_ref[...] = jnp.zeros_like(o_ref)

    x = x_ref[...]
    sp = jnp.maximum(x, 0.0) + jnp.log1p(jnp.exp(-jnp.abs(x)))
    sp_sum = jnp.sum(sp)

    W = x.shape[1]
    r = pltpu.roll(x, W - 1, 1)
    m1 = jnp.maximum(x, r)
    m2 = jnp.minimum(x, r)
    for s in (2, 4, 8, 16, 32):
        r1 = pltpu.roll(m1, W - s, 1)
        r2 = pltpu.roll(m2, W - s, 1)
        m2 = jnp.maximum(jnp.minimum(m1, r1), jnp.maximum(m2, r2))
        m1 = jnp.maximum(m1, r1)
    lane = jax.lax.broadcasted_iota(jnp.int32, x.shape, 1)
    pick = ((lane & 63) == 0) & (m1 > 0.0)
    contrib = jnp.where(pick, m1 + m2, 0.0)
    o_ref[...] += sp_sum - jnp.sum(contrib)


def kernel(pred_logits):
    N, T = pred_logits.shape
    grid = N // _BR
    out = pl.pallas_call(
        _body,
        grid=(grid,),
        in_specs=[pl.BlockSpec((_BR, T), lambda i: (i, 0))],
        out_specs=pl.BlockSpec((1, 1), lambda i: (0, 0)),
        out_shape=jax.ShapeDtypeStruct((1, 1), jnp.float32),
    )(pred_logits)
    return out[0, 0]


# speedup vs baseline: 13.9458x; 1.0155x over previous
import jax
import jax.numpy as jnp
from jax.experimental import pallas as pl
from jax.experimental.pallas import tpu as pltpu

_BR = 256


_RC = 8
_CC = 512


def _tile_partial(t, mask0):
    sp = jnp.maximum(t, 0.0) + jnp.log1p(jnp.exp(-jnp.abs(t)))
    r = pltpu.roll(t, _CC - 1, 1)
    m1 = jnp.maximum(t, r)
    m2 = jnp.minimum(t, r)
    for s in (2, 4, 8, 16, 32):
        r1 = pltpu.roll(m1, _CC - s, 1)
        r2 = pltpu.roll(m2, _CC - s, 1)
        m2 = jnp.maximum(jnp.minimum(m1, r1), jnp.maximum(m2, r2))
        m1 = jnp.maximum(m1, r1)
    contrib = jnp.where(mask0 & (m1 > 0.0), m1 + m2, 0.0)
    return sp - contrib


def _body(x_ref, o_ref):
    @pl.when(pl.program_id(0) == 0)
    def _init():
        o_ref[...] = jnp.zeros_like(o_ref)

    lane = jax.lax.broadcasted_iota(jnp.int32, (_RC, _CC), 1)
    mask0 = (lane & 63) == 0
    ncc = x_ref.shape[1] // _CC

    def row_step(i, acc):
        for c in range(ncc):
            t = x_ref[pl.ds(i * _RC, _RC), pl.ds(c * _CC, _CC)]
            acc = acc + _tile_partial(t, mask0)
        return acc

    acc = jax.lax.fori_loop(
        0, x_ref.shape[0] // _RC, row_step,
        jnp.zeros((_RC, _CC), jnp.float32))
    o_ref[...] += jnp.sum(acc)


def kernel(pred_logits):
    N, T = pred_logits.shape
    grid = N // _BR
    out = pl.pallas_call(
        _body,
        grid=(grid,),
        in_specs=[pl.BlockSpec((_BR, T), lambda i: (i, 0))],
        out_specs=pl.BlockSpec((1, 1), lambda i: (0, 0)),
        out_shape=jax.ShapeDtypeStruct((1, 1), jnp.float32),
    )(pred_logits)
    return out[0, 0]
